# Initial kernel scaffold; baseline (speedup 1.0000x reference)
import jax
import jax.numpy as jnp
from jax.experimental import pallas as pl


def kernel(x):
    raise NotImplementedError("write your pallas kernel here")


# speedup vs baseline: 1.0281x; 1.0281x over previous
import jax
import jax.numpy as jnp
from jax.experimental import pallas as pl

_P = 16
_RATIO = 0.75


def _pix_mask(hb, wb):
    L = hb * wb
    masked_num = int(L * _RATIO)
    idx = jax.random.permutation(jax.random.key(42), L)
    keep = (idx >= masked_num).reshape(hb, wb).astype(jnp.float32)
    return jnp.kron(keep, jnp.ones((_P, _P), jnp.float32))


def _body(x_ref, m_ref, o_ref):
    o_ref[...] = x_ref[...] * m_ref[...][None]


def kernel(x):
    B, C, H, W = x.shape
    hb, wb = H // _P, W // _P
    xf = x.reshape(B * C, H, W)
    pix = _pix_mask(hb, wb)
    G = 16
    out = pl.pallas_call(
        _body,
        grid=(B * C // G,),
        in_specs=[
            pl.BlockSpec((G, H, W), lambda i: (i, 0, 0)),
            pl.BlockSpec((H, W), lambda i: (0, 0)),
        ],
        out_specs=pl.BlockSpec((G, H, W), lambda i: (i, 0, 0)),
        out_shape=jax.ShapeDtypeStruct((B * C, H, W), x.dtype),
    )(xf, pix)
    return out.reshape(B, C, H, W)
